# baseline (speedup 1.0000x reference)
import functools

import numpy as np
import jax
import jax.numpy as jnp
from jax import lax
from jax.experimental import pallas as pl
from jax.experimental.pallas import tpu as pltpu
from jax.experimental.pallas import tpu_sc as plsc

N = 8192
G = 256
NPG = 32
EDG = 32768
D = 256
DE = 256
NV = 64
AC = 16
EC = 5
HEADS = 4
LAYERS = 4
HD = D // HEADS

NC = 2
NS = 16
NW = NC * NS

BN = 512
BE = 512

F32 = jnp.float32


def _mm(a, b):
    return lax.dot_general(a, b, (((1,), (0,)), ((), ())),
                           preferred_element_type=F32)


def _ln(x):
    m = jnp.mean(x, -1, keepdims=True)
    v = jnp.mean((x - m) ** 2, -1, keepdims=True)
    return (x - m) * lax.rsqrt(v + 1e-6)


def _silu(x):
    return x * jax.nn.sigmoid(x)


def _b2(x):
    return x.reshape(1, -1)


_MESH = plsc.VectorSubcoreMesh(core_axis_name="c", subcore_axis_name="s",
                               num_cores=NC, num_subcores=NS)


@functools.cache
def _gather_kernel(widths, srcs, ch):
    CH = ch
    k = len(widths)
    perw = EDG // NW
    nch = perw // CH

    out_type = [jax.ShapeDtypeStruct((EDG, w), F32) for w in widths]
    scratch = ([pltpu.VMEM((perw,), jnp.int32), pltpu.VMEM((perw,), jnp.int32)]
               + [pltpu.VMEM((CH, w), F32) for w in widths for _ in (0, 1)]
               + [pltpu.SemaphoreType.DMA] * 4)

    def body(*refs):
        tabs = refs[:k]
        rowi, coli = refs[k], refs[k + 1]
        outs = refs[k + 2:2 * k + 2]
        irb, icb = refs[2 * k + 2], refs[2 * k + 3]
        bufs = refs[2 * k + 4:2 * k + 4 + 2 * k]
        gsem = refs[4 * k + 4:4 * k + 6]
        osem = refs[4 * k + 6:4 * k + 8]
        wid = lax.axis_index("s") * NC + lax.axis_index("c")
        e0 = pl.multiple_of(wid * perw, perw)
        pltpu.sync_copy(rowi.at[pl.ds(e0, perw)], irb)
        pltpu.sync_copy(coli.at[pl.ds(e0, perw)], icb)

        def iref(t, j):
            b = irb if srcs[t] == 0 else icb
            return b.at[pl.ds(j * CH, CH)]

        def issue_gathers(j, s):
            for t in range(k):
                pltpu.async_copy(tabs[t].at[iref(t, j)], bufs[2 * t + s],
                                 gsem[s])

        def wait_gathers(s):
            for t in range(k):
                pltpu.make_async_copy(tabs[t].at[iref(t, 0)],
                                      bufs[2 * t + s], gsem[s]).wait()

        def issue_outs(j, s):
            for t in range(k):
                pltpu.async_copy(bufs[2 * t + s],
                                 outs[t].at[pl.ds(e0 + j * CH, CH)], osem[s])

        def wait_outs(s):
            for t in range(k):
                pltpu.make_async_copy(bufs[2 * t + s],
                                      outs[t].at[pl.ds(e0, CH)],
                                      osem[s]).wait()

        issue_gathers(0, 0)

        def pair(t2, carry):
            for u in (0, 1):
                s, s2 = u, 1 - u
                j = 2 * t2 + u

                @pl.when(j >= 1)
                def _():
                    wait_outs(s2)

                @pl.when(j + 1 < nch)
                def _():
                    issue_gathers(j + 1, s2)

                wait_gathers(s)
                issue_outs(j, s)
            return carry

        lax.fori_loop(0, nch // 2, pair, 0)
        wait_outs(1)

    return pl.kernel(body, out_type=out_type, mesh=_MESH,
                     scratch_types=scratch)


@functools.cache
def _scatter_kernel(ngrp):
    CH = 128
    WC = 128
    gpc = ngrp // 2
    rps = N // NS
    eps = EDG // NS
    nch = eps // CH

    out_type = jax.ShapeDtypeStruct((ngrp, N, WC), F32)
    scratch = [
        pltpu.VMEM((CH,), jnp.int32), pltpu.VMEM((CH,), jnp.int32),
        pltpu.VMEM((CH, WC), F32), pltpu.VMEM((CH, WC), F32),
        pltpu.VMEM_SHARED((N, WC), F32),
        pltpu.SemaphoreType.DMA, pltpu.SemaphoreType.DMA,
    ]

    def body(src, idx, zin, out, iv0, iv1, sb0, sb1, acc, ls0, ls1):
        iv = (iv0, iv1)
        sb = (sb0, sb1)
        ls = (ls0, ls1)
        ci = lax.axis_index("c")
        sid = lax.axis_index("s")
        r0 = pl.multiple_of(sid * rps, rps)
        e0 = pl.multiple_of(sid * eps, eps)

        for gp in range(gpc):
            g = ci * gpc + gp

            def issue_load(j, s):
                pltpu.async_copy(idx.at[pl.ds(e0 + j * CH, CH)], iv[s],
                                 ls[s])
                pltpu.async_copy(src.at[g, pl.ds(e0 + j * CH, CH)], sb[s],
                                 ls[s])

            def wait_load(s):
                pltpu.make_async_copy(idx.at[pl.ds(e0, CH)], iv[s],
                                      ls[s]).wait()
                pltpu.make_async_copy(src.at[g, pl.ds(e0, CH)], sb[s],
                                      ls[s]).wait()

            issue_load(0, 0)
            pltpu.sync_copy(zin, acc.at[pl.ds(r0, rps)])
            plsc.subcore_barrier()

            def pair(t2, carry):
                for u in (0, 1):
                    s, s2 = u, 1 - u
                    j = 2 * t2 + u

                    @pl.when(j + 1 < nch)
                    def _():
                        issue_load(j + 1, s2)

                    wait_load(s)
                    pltpu.sync_copy(sb[s], acc.at[iv[s]], add=True)
                return carry

            lax.fori_loop(0, nch // 2, pair, 0)
            plsc.subcore_barrier()
            pltpu.sync_copy(acc.at[pl.ds(r0, rps)],
                            out.at[g, pl.ds(r0, rps)])

    return pl.kernel(body, out_type=out_type, mesh=_MESH,
                     scratch_types=scratch)


def _seg_sum4(src4, idx):
    zin = jnp.zeros((N // NS, 128), F32)
    return _scatter_kernel(4)(src4, idx, zin)


def _seg_sum2(src2, idx):
    zin = jnp.zeros((N // NS, 128), F32)
    out = _scatter_kernel(2)(src2, idx, zin)
    return jnp.concatenate([out[0], out[1]], 1)


def _rows(bm, w):
    return pl.BlockSpec((bm, w), lambda i: (i, 0))


def _rep(shape):
    return pl.BlockSpec(shape, lambda i: tuple(0 for _ in shape))


def _te_body(t8, freqs, w1, b1, w2, b2, out):
    args = t8[:, 0:1] * freqs[...]
    te0 = jnp.concatenate([jnp.cos(args), jnp.sin(args)], 1)
    out[...] = _mm(_silu(_mm(te0, w1[...]) + b1[...]), w2[...]) + b2[...]


def _prep_body(hp, xp, w1, b1, w2, b2, cw, hout, pout):
    hout[...] = _mm(_silu(_mm(hp[...], w1[...]) + b1[...]), w2[...]) + b2[...]
    x = xp[...]
    pout[...] = jnp.concatenate(
        [x[:, c:c + 1] * cw[...] for c in range(3)]
        + [jnp.zeros((BN, 64), F32)], 1)


def _dual_body(x, wa, wb, aout, bout):
    xv = x[...]
    aout[...] = _mm(xv, wa[...])
    bout[...] = _mm(xv, wb[...])


def _edge_in_body(pr, pc, ar, bc, ep, w1e, b1e, w2e, b2e, wef, wd, b1m,
                  w2m, b2m, wphi, bphi, s4):
    d = pr[...] - pc[...]
    dist = (d[:, 0:64] ** 2 + d[:, 64:128] ** 2 + d[:, 128:192] ** 2)
    xe = jnp.concatenate([ep[...], dist], 1)
    ef = _mm(_silu(_mm(xe, w1e[...]) + b1e[...]), w2e[...]) + b2e[...]
    pre = ar[...] + bc[...] + _mm(ef, wef[...]) + _mm(dist, wd[...]) + b1m[...]
    m = _mm(_silu(pre), w2m[...]) + b2m[...]
    phi = _mm(m, wphi[...]) + bphi[...]
    s4[0, :, :] = m[:, 0:128]
    s4[1, :, :] = m[:, 128:256]
    s4[2, :, :] = jnp.concatenate([d[:, 0:64] * phi, d[:, 64:128] * phi], 1)
    s4[3, :, :] = jnp.concatenate([d[:, 128:192] * phi,
                                   jnp.ones((BE, 64), F32)], 1)


def _edge_layer_body(ar, bc, xr, xc, wrad, b1, w2, b2, wphi, bphi, s4):
    d = xr[...] - xc[...]
    rad = (d[:, 0:64] ** 2 + d[:, 64:128] ** 2 + d[:, 128:192] ** 2)
    pre = ar[...] + bc[...] + _mm(rad, wrad[...]) + b1[...]
    m = _mm(_silu(pre), w2[...]) + b2[...]
    phi = _mm(m, wphi[...]) + bphi[...]
    s4[0, :, :] = m[:, 0:128]
    s4[1, :, :] = m[:, 128:256]
    s4[2, :, :] = jnp.concatenate([d[:, 0:64] * phi, d[:, 64:128] * phi], 1)
    s4[3, :, :] = jnp.concatenate([d[:, 128:192] * phi,
                                   jnp.zeros((BE, 64), F32)], 1)


def _edge_out_body(ar, bc, xr, xc, wrad, b1, w2, b2, wp8, ea2, d1_out):
    d = xr[...] - xc[...]
    rad = (d[:, 0:64] ** 2 + d[:, 64:128] ** 2 + d[:, 128:192] ** 2)
    pre = ar[...] + bc[...] + _mm(rad, wrad[...]) + b1[...]
    ea = _mm(_silu(pre), w2[...]) + b2[...]
    ea2[0, :, :] = ea[:, 0:128]
    ea2[1, :, :] = ea[:, 128:256]
    dc0 = _mm(d[:, 0:64], wp8[...])
    dc1 = _mm(d[:, 64:128], wp8[...])
    dc2 = _mm(d[:, 128:192], wp8[...])
    d1_out[...] = dc0 ** 2 + dc1 ** 2 + dc2 ** 2


def _upd_in_body(h, pos, a4, wu1, wu2, bu, hout, xout, cout):
    am = jnp.concatenate([a4[0, :, :], a4[1, :, :]], 1)
    ad = jnp.concatenate([a4[2, :, :], a4[3, :, :]], 1)
    hv = h[...]
    cnt = ad[:, 192:193] + 1.0
    hout[...] = hv + _mm(hv, wu1[...]) + _mm(am, wu2[...]) + bu[...]
    xout[...] = pos[...] + ad / cnt
    cout[...] = jnp.broadcast_to(cnt, (BN, 8))


def _upd_layer_body(h, xv, a4, cnt, wu1, wu2, bu, hout, xout):
    am = jnp.concatenate([a4[0, :, :], a4[1, :, :]], 1)
    ad = jnp.concatenate([a4[2, :, :], a4[3, :, :]], 1)
    hv = h[...]
    hout[...] = hv + _mm(hv, wu1[...]) + _mm(am, wu2[...]) + bu[...]
    xout[...] = xv[...] + ad / cnt[:, 0:1]


def _mod_body(te, wada, bada, out):
    out[...] = _mm(_silu(te[...]), wada[...]) + bada[...]


def _dit_body(h_ref, mod_ref, wqkv, bqkv, wo, bo, w1, b1, w2, b2, out):
    BM = 256
    GPB = BM // NPG
    H = h_ref[...]
    modb = mod_ref[...]
    rg = lax.broadcasted_iota(jnp.int32, (BM, GPB), 0) // NPG
    cg = lax.broadcasted_iota(jnp.int32, (BM, GPB), 1)
    R = (rg == cg).astype(F32)
    modx = _mm(R, modb)
    sa = modx[:, 0:256]
    ca = modx[:, 256:512]
    ga = modx[:, 512:768]
    sm = modx[:, 768:1024]
    cm = modx[:, 1024:1280]
    gm = modx[:, 1280:1536]
    h = _ln(H) * (1.0 + ca) + sa
    qkv = _mm(h, wqkv[...]) + bqkv[...]
    rr = lax.broadcasted_iota(jnp.int32, (BM, BM), 0) // NPG
    cc = lax.broadcasted_iota(jnp.int32, (BM, BM), 1) // NPG
    blockmask = rr == cc
    outs = []
    for hh in range(HEADS):
        q = qkv[:, hh * HD:(hh + 1) * HD]
        k = qkv[:, 256 + hh * HD:256 + (hh + 1) * HD]
        v = qkv[:, 512 + hh * HD:512 + (hh + 1) * HD]
        s = lax.dot_general(q, k, (((1,), (1,)), ((), ())),
                            preferred_element_type=F32) * (1.0 / 8.0)
        s = jnp.where(blockmask, s, -1e30)
        e = jnp.exp(s - jnp.max(s, -1, keepdims=True))
        att = e / jnp.sum(e, -1, keepdims=True)
        outs.append(_mm(att, v))
    o = jnp.concatenate(outs, 1)
    H2 = H + ga * (_mm(o, wo[...]) + bo[...])
    h2 = _ln(H2) * (1.0 + cm) + sm
    y = _mm(_silu(_mm(h2, w1[...]) + b1[...]), w2[...]) + b2[...]
    out[...] = H2 + gm * y


def _out_node_body(h, xv, agg, wu1, wu2, bu, wp8, wa1, ba1, wa2, ba2,
                   hout, xcout, hlout):
    hv = h[...]
    h2 = hv + _mm(hv, wu1[...]) + _mm(agg[...], wu2[...]) + bu[...]
    hout[...] = h2
    x = xv[...]
    xp = jnp.concatenate([_mm(x[:, c * 64:(c + 1) * 64], wp8[...])
                          for c in range(3)], 1)
    rr = lax.broadcasted_iota(jnp.int32, (BN, BN), 0) // NPG
    cc = lax.broadcasted_iota(jnp.int32, (BN, BN), 1) // NPG
    avg = (rr == cc).astype(F32) * (1.0 / NPG)
    xcout[...] = xp - _mm(avg, xp)
    hlout[...] = _mm(_silu(_mm(h2, wa1[...]) + ba1[...]), wa2[...]) + ba2[...]


def _refine_body(a2, b2r, ea, d1, wea, wd18, b1, w2, b2, weh, beh, out):
    eav = jnp.concatenate([ea[0, :, :], ea[1, :, :]], 1)
    pre = (a2[...] + b2r[...] + _mm(eav, wea[...]) + _mm(d1[...], wd18[...])
           + b1[...])
    delta = _mm(_silu(pre), w2[...]) + b2[...]
    eaf = eav + delta
    out[...] = _mm(eaf, weh[...]) + beh[...]


def _call(body, grid, in_specs, out_specs, out_shape):
    return pl.pallas_call(body, grid=grid, in_specs=in_specs,
                          out_specs=out_specs, out_shape=out_shape)


@jax.jit
def kernel(batch, X, H, E_idx, E, t, params):
    row = E_idx[0].astype(jnp.int32)
    col = E_idx[1].astype(jnp.int32)
    p = params

    def wb(lp):
        return lp["w"], _b2(lp["b"])

    ae1w, ae1b = wb(p["atom_emb"]["l1"])
    ae2w, ae2b = wb(p["atom_emb"]["l2"])
    ae1w = jnp.pad(ae1w, ((0, 128 - AC), (0, 0)))
    Hp = jnp.pad(H, ((0, 0), (0, 128 - AC)))
    Xp8 = jnp.pad(X, ((0, 0), (0, 8 - 3)))
    cw = _b2(p["coord_emb_w"])

    ee1w, ee1b = wb(p["edge_emb"]["l1"])
    ee2w, ee2b = wb(p["edge_emb"]["l2"])
    w1e_pad = jnp.zeros((128, DE), F32)
    w1e_pad = w1e_pad.at[0:EC].set(ee1w[0:EC]).at[64:64 + NV].set(ee1w[EC:])
    Epad = jnp.pad(E, ((0, 0), (0, 64 - EC)))

    tm1w, tm1b = wb(p["time_mlp"]["l1"])
    tm2w, tm2b = wb(p["time_mlp"]["l2"])
    freqs = _b2(jnp.exp(-np.log(10000.0)
                        * jnp.arange(D // 2, dtype=F32) / (D // 2)))
    t8 = jnp.broadcast_to(t[:, None], (G, 8))

    im1w, im1b = wb(p["in_msg"]["l1"])
    im2w, im2b = wb(p["in_msg"]["l2"])
    im_wa, im_wb = im1w[0:D], im1w[D:2 * D]
    im_wef, im_wd = im1w[2 * D:2 * D + DE], im1w[2 * D + DE:]
    iu_w, iu_b = wb(p["in_upd"])
    ip_w, ip_b = wb(p["in_phi"])

    wp8 = jnp.broadcast_to(p["coord_pred_w"][:, None], (NV, 8))

    oe1w, oe1b = wb(p["out_edge"]["l1"])
    oe2w, oe2b = wb(p["out_edge"]["l2"])
    ou_w, ou_b = wb(p["out_upd"])
    rf1w, rf1b = wb(p["refine"]["l1"])
    rf2w, rf2b = wb(p["refine"]["l2"])
    rf_wa, rf_wb = rf1w[0:D], rf1w[D:2 * D]
    rf_wea = rf1w[2 * D:2 * D + DE]
    rf_wd18 = jnp.broadcast_to(rf1w[2 * D + DE:2 * D + DE + 1] / 8.0, (8, DE))
    ah1w, ah1b = wb(p["atom_head"]["l1"])
    ah2w, ah2b = wb(p["atom_head"]["l2"])
    eh_w, eh_b = wb(p["edge_head"])
    eh_wp = jnp.pad(eh_w, ((0, 0), (0, 8 - EC)))
    eh_bp = jnp.pad(eh_b, ((0, 0), (0, 8 - EC)))

    gn = N // BN
    ge = EDG // BE

    te_g = _call(
        _te_body, (1,),
        [_rep((G, 8)), _rep((1, 128)), _rep((D, D)), _rep((1, D)),
         _rep((D, D)), _rep((1, D))],
        _rep((G, D)), jax.ShapeDtypeStruct((G, D), F32),
    )(t8, freqs, tm1w, tm1b, tm2w, tm2b)

    Hf, pos = _call(
        _prep_body, (gn,),
        [_rows(BN, 128), _rows(BN, 8), _rep((128, D)), _rep((1, D)),
         _rep((D, D)), _rep((1, D)), _rep((1, NV))],
        [_rows(BN, D), _rows(BN, 256)],
        [jax.ShapeDtypeStruct((N, D), F32),
         jax.ShapeDtypeStruct((N, 256), F32)],
    )(Hp, Xp8, ae1w, ae1b, ae2w, ae2b, cw)

    def dual(x, wa, wbm):
        return _call(
            _dual_body, (gn,),
            [_rows(BN, D), _rep((D, D)), _rep((D, D))],
            [_rows(BN, D), _rows(BN, D)],
            [jax.ShapeDtypeStruct((N, D), F32),
             jax.ShapeDtypeStruct((N, D), F32)],
        )(x, wa, wbm)

    A, B = dual(Hf, im_wa, im_wb)
    Ar, Bc, Pr, Pc = _gather_kernel((D, D, 256, 256), (0, 1, 0, 1),
                                    32)(A, B, pos, pos, row, col)
    S4 = _call(
        _edge_in_body, (ge,),
        [_rows(BE, 256), _rows(BE, 256), _rows(BE, D), _rows(BE, D),
         _rows(BE, 64), _rep((128, DE)), _rep((1, DE)), _rep((DE, DE)),
         _rep((1, DE)), _rep((DE, D)), _rep((NV, D)), _rep((1, D)),
         _rep((D, D)), _rep((1, D)), _rep((D, NV)), _rep((1, NV))],
        pl.BlockSpec((4, BE, 128), lambda i: (0, i, 0)),
        jax.ShapeDtypeStruct((4, EDG, 128), F32),
    )(Pr, Pc, Ar, Bc, Epad, w1e_pad, ee1b, ee2w, ee2b, im_wef, im_wd,
      im1b, im2w, im2b, ip_w, ip_b)
    agg4 = _seg_sum4(S4, row)
    Hf, Xv, cnt = _call(
        _upd_in_body, (gn,),
        [_rows(BN, D), _rows(BN, 256),
         pl.BlockSpec((4, BN, 128), lambda i: (0, i, 0)),
         _rep((D, D)), _rep((D, D)), _rep((1, D))],
        [_rows(BN, D), _rows(BN, 256), _rows(BN, 8)],
        [jax.ShapeDtypeStruct((N, D), F32),
         jax.ShapeDtypeStruct((N, 256), F32),
         jax.ShapeDtypeStruct((N, 8), F32)],
    )(Hf, pos, agg4, iu_w[0:D], iu_w[D:], iu_b)

    for i in range(LAYERS):
        dp = p["dit"][i]
        ep = p["egnn"][i]
        ada_w, ada_b = wb(dp["ada"])
        qkv_w, qkv_b = wb(dp["qkv"])
        wo_w, wo_b = wb(dp["wo"])
        dm1w, dm1b = wb(dp["mlp"]["l1"])
        dm2w, dm2b = wb(dp["mlp"]["l2"])
        mg1w, mg1b = wb(ep["msg"]["l1"])
        mg2w, mg2b = wb(ep["msg"]["l2"])
        up_w, up_b = wb(ep["upd"])
        ph_w, ph_b = wb(ep["phi"])

        Xr, Xc = _gather_kernel((256, 256), (0, 1), 64)(Xv, Xv, row, col)

        mod_g = _call(
            _mod_body, (1,),
            [_rep((G, D)), _rep((D, 6 * D)), _rep((1, 6 * D))],
            _rep((G, 6 * D)), jax.ShapeDtypeStruct((G, 6 * D), F32),
        )(te_g, ada_w, ada_b)

        Hd = _call(
            _dit_body, (N // 256,),
            [_rows(256, D), pl.BlockSpec((8, 6 * D), lambda i: (i, 0)),
             _rep((D, 3 * D)), _rep((1, 3 * D)), _rep((D, D)), _rep((1, D)),
             _rep((D, 4 * D)), _rep((1, 4 * D)), _rep((4 * D, D)),
             _rep((1, D))],
            _rows(256, D), jax.ShapeDtypeStruct((N, D), F32),
        )(Hf, mod_g, qkv_w, qkv_b, wo_w, wo_b, dm1w, dm1b, dm2w, dm2b)

        A, B = dual(Hd, mg1w[0:D], mg1w[D:2 * D])
        Ar, Bc = _gather_kernel((D, D), (0, 1), 64)(A, B, row, col)
        S4 = _call(
            _edge_layer_body, (ge,),
            [_rows(BE, D), _rows(BE, D), _rows(BE, 256), _rows(BE, 256),
             _rep((NV, D)), _rep((1, D)), _rep((D, D)), _rep((1, D)),
             _rep((D, NV)), _rep((1, NV))],
            pl.BlockSpec((4, BE, 128), lambda i: (0, i, 0)),
            jax.ShapeDtypeStruct((4, EDG, 128), F32),
        )(Ar, Bc, Xr, Xc, mg1w[2 * D:], mg1b, mg2w, mg2b, ph_w, ph_b)
        agg4 = _seg_sum4(S4, row)
        Hf, Xv = _call(
            _upd_layer_body, (gn,),
            [_rows(BN, D), _rows(BN, 256),
             pl.BlockSpec((4, BN, 128), lambda i: (0, i, 0)),
             _rows(BN, 8), _rep((D, D)), _rep((D, D)), _rep((1, D))],
            [_rows(BN, D), _rows(BN, 256)],
            [jax.ShapeDtypeStruct((N, D), F32),
             jax.ShapeDtypeStruct((N, 256), F32)],
        )(Hd, Xv, agg4, cnt, up_w[0:D], up_w[D:], up_b)

    A, B = dual(Hf, oe1w[0:D], oe1w[D:2 * D])
    Ar, Bc, Xr, Xc = _gather_kernel((D, D, 256, 256), (0, 1, 0, 1),
                                    32)(A, B, Xv, Xv, row, col)
    ea2, d1 = _call(
        _edge_out_body, (ge,),
        [_rows(BE, D), _rows(BE, D), _rows(BE, 256), _rows(BE, 256),
         _rep((NV, DE)), _rep((1, DE)), _rep((DE, DE)), _rep((1, DE)),
         _rep((NV, 8))],
        [pl.BlockSpec((2, BE, 128), lambda i: (0, i, 0)), _rows(BE, 8)],
        [jax.ShapeDtypeStruct((2, EDG, 128), F32),
         jax.ShapeDtypeStruct((EDG, 8), F32)],
    )(Ar, Bc, Xr, Xc, oe1w[2 * D:], oe1b, oe2w, oe2b, wp8)
    agg = _seg_sum2(ea2, row)
    Hf2, xc24, hlog = _call(
        _out_node_body, (gn,),
        [_rows(BN, D), _rows(BN, 256), _rows(BN, D), _rep((D, D)),
         _rep((D, D)), _rep((1, D)), _rep((NV, 8)), _rep((D, D)),
         _rep((1, D)), _rep((D, AC)), _rep((1, AC))],
        [_rows(BN, D), _rows(BN, 24), _rows(BN, AC)],
        [jax.ShapeDtypeStruct((N, D), F32),
         jax.ShapeDtypeStruct((N, 24), F32),
         jax.ShapeDtypeStruct((N, AC), F32)],
    )(Hf, Xv, agg, ou_w[0:D], ou_w[D:], ou_b, wp8, ah1w, ah1b, ah2w, ah2b)

    A2, B2 = dual(Hf2, rf_wa, rf_wb)
    A2r, B2c = _gather_kernel((D, D), (0, 1), 64)(A2, B2, row, col)
    elog = _call(
        _refine_body, (ge,),
        [_rows(BE, D), _rows(BE, D),
         pl.BlockSpec((2, BE, 128), lambda i: (0, i, 0)), _rows(BE, 8),
         _rep((DE, DE)), _rep((8, DE)), _rep((1, DE)), _rep((DE, DE)),
         _rep((1, DE)), _rep((DE, 8)), _rep((1, 8))],
        _rows(BE, 8), jax.ShapeDtypeStruct((EDG, 8), F32),
    )(A2r, B2c, ea2, d1, rf_wea, rf_wd18, rf1b, rf2w, rf2b, eh_wp, eh_bp)

    x = jnp.stack([xc24[:, 0], xc24[:, 8], xc24[:, 16]], 1)
    return x, hlog, elog[:, :EC]


# speedup vs baseline: 1.0747x; 1.0747x over previous
import functools

import numpy as np
import jax
import jax.numpy as jnp
from jax import lax
from jax.experimental import pallas as pl
from jax.experimental.pallas import tpu as pltpu
from jax.experimental.pallas import tpu_sc as plsc

N = 8192
G = 256
NPG = 32
EDG = 32768
D = 256
DE = 256
NV = 64
AC = 16
EC = 5
HEADS = 4
LAYERS = 4
HD = D // HEADS

NC = 2
NS = 16
NW = NC * NS

BN = 512
BE = 512

F32 = jnp.float32


def _mm(a, b):
    return lax.dot_general(a, b, (((1,), (0,)), ((), ())),
                           preferred_element_type=F32)


def _ln(x):
    m = jnp.mean(x, -1, keepdims=True)
    v = jnp.mean((x - m) ** 2, -1, keepdims=True)
    return (x - m) * lax.rsqrt(v + 1e-6)


def _silu(x):
    return x * jax.nn.sigmoid(x)


def _b2(x):
    return x.reshape(1, -1)


_MESH = plsc.VectorSubcoreMesh(core_axis_name="c", subcore_axis_name="s",
                               num_cores=NC, num_subcores=NS)


@functools.cache
def _gather_kernel(widths, srcs, ch):
    CH = ch
    k = len(widths)
    perw = EDG // NW
    nch = perw // CH

    out_type = [jax.ShapeDtypeStruct((EDG, w), F32) for w in widths]
    scratch = ([pltpu.VMEM((perw,), jnp.int32), pltpu.VMEM((perw,), jnp.int32)]
               + [pltpu.VMEM((CH, w), F32) for w in widths for _ in (0, 1)]
               + [pltpu.SemaphoreType.DMA] * 4)

    def body(*refs):
        tabs = refs[:k]
        rowi, coli = refs[k], refs[k + 1]
        outs = refs[k + 2:2 * k + 2]
        irb, icb = refs[2 * k + 2], refs[2 * k + 3]
        bufs = refs[2 * k + 4:2 * k + 4 + 2 * k]
        gsem = refs[4 * k + 4:4 * k + 6]
        osem = refs[4 * k + 6:4 * k + 8]
        wid = lax.axis_index("s") * NC + lax.axis_index("c")
        e0 = pl.multiple_of(wid * perw, perw)
        pltpu.sync_copy(rowi.at[pl.ds(e0, perw)], irb)
        pltpu.sync_copy(coli.at[pl.ds(e0, perw)], icb)

        def iref(t, j):
            b = irb if srcs[t] == 0 else icb
            return b.at[pl.ds(j * CH, CH)]

        def issue_gathers(j, s):
            for t in range(k):
                pltpu.async_copy(tabs[t].at[iref(t, j)], bufs[2 * t + s],
                                 gsem[s])

        def wait_gathers(s):
            for t in range(k):
                pltpu.make_async_copy(tabs[t].at[iref(t, 0)],
                                      bufs[2 * t + s], gsem[s]).wait()

        def issue_outs(j, s):
            for t in range(k):
                pltpu.async_copy(bufs[2 * t + s],
                                 outs[t].at[pl.ds(e0 + j * CH, CH)], osem[s])

        def wait_outs(s):
            for t in range(k):
                pltpu.make_async_copy(bufs[2 * t + s],
                                      outs[t].at[pl.ds(e0, CH)],
                                      osem[s]).wait()

        issue_gathers(0, 0)

        def pair(t2, carry):
            for u in (0, 1):
                s, s2 = u, 1 - u
                j = 2 * t2 + u

                @pl.when(j >= 1)
                def _():
                    wait_outs(s2)

                @pl.when(j + 1 < nch)
                def _():
                    issue_gathers(j + 1, s2)

                wait_gathers(s)
                issue_outs(j, s)
            return carry

        lax.fori_loop(0, nch // 2, pair, 0)
        wait_outs(1)

    return pl.kernel(body, out_type=out_type, mesh=_MESH,
                     scratch_types=scratch)


@functools.cache
def _scatter_kernel(ngrp):
    CH = 128
    WC = 128
    gpc = ngrp // 2
    rps = N // NS
    eps = EDG // NS
    nch = eps // CH

    out_type = jax.ShapeDtypeStruct((ngrp, N, WC), F32)
    scratch = [
        pltpu.VMEM((CH,), jnp.int32), pltpu.VMEM((CH,), jnp.int32),
        pltpu.VMEM((CH, WC), F32), pltpu.VMEM((CH, WC), F32),
        pltpu.VMEM_SHARED((N, WC), F32),
        pltpu.SemaphoreType.DMA, pltpu.SemaphoreType.DMA,
    ]

    def body(src, idx, zin, out, iv0, iv1, sb0, sb1, acc, ls0, ls1):
        iv = (iv0, iv1)
        sb = (sb0, sb1)
        ls = (ls0, ls1)
        ci = lax.axis_index("c")
        sid = lax.axis_index("s")
        r0 = pl.multiple_of(sid * rps, rps)
        e0 = pl.multiple_of(sid * eps, eps)

        for gp in range(gpc):
            g = ci * gpc + gp

            def issue_load(j, s):
                pltpu.async_copy(idx.at[pl.ds(e0 + j * CH, CH)], iv[s],
                                 ls[s])
                pltpu.async_copy(src.at[g, pl.ds(e0 + j * CH, CH)], sb[s],
                                 ls[s])

            def wait_load(s):
                pltpu.make_async_copy(idx.at[pl.ds(e0, CH)], iv[s],
                                      ls[s]).wait()
                pltpu.make_async_copy(src.at[g, pl.ds(e0, CH)], sb[s],
                                      ls[s]).wait()

            issue_load(0, 0)
            pltpu.sync_copy(zin, acc.at[pl.ds(r0, rps)])
            plsc.subcore_barrier()

            def pair(t2, carry):
                for u in (0, 1):
                    s, s2 = u, 1 - u
                    j = 2 * t2 + u

                    @pl.when(j + 1 < nch)
                    def _():
                        issue_load(j + 1, s2)

                    wait_load(s)
                    pltpu.sync_copy(sb[s], acc.at[iv[s]], add=True)
                return carry

            lax.fori_loop(0, nch // 2, pair, 0)
            plsc.subcore_barrier()
            pltpu.sync_copy(acc.at[pl.ds(r0, rps)],
                            out.at[g, pl.ds(r0, rps)])

    return pl.kernel(body, out_type=out_type, mesh=_MESH,
                     scratch_types=scratch)


def _seg_sum4(src4, idx):
    zin = jnp.zeros((N // NS, 128), F32)
    return _scatter_kernel(4)(src4, idx, zin)


def _seg_sum2(src2, idx):
    zin = jnp.zeros((N // NS, 128), F32)
    out = _scatter_kernel(2)(src2, idx, zin)
    return jnp.concatenate([out[0], out[1]], 1)


def _rows(bm, w):
    return pl.BlockSpec((bm, w), lambda i: (i, 0))


def _rep(shape):
    return pl.BlockSpec(shape, lambda i: tuple(0 for _ in shape))


def _te_body(t8, freqs, w1, b1, w2, b2, out):
    args = t8[:, 0:1] * freqs[...]
    te0 = jnp.concatenate([jnp.cos(args), jnp.sin(args)], 1)
    out[...] = _mm(_silu(_mm(te0, w1[...]) + b1[...]), w2[...]) + b2[...]


def _prep_body(hp, xp, w1, b1, w2, b2, cw, wa, wbm, hout, pout, aout, bout):
    hf = _mm(_silu(_mm(hp[...], w1[...]) + b1[...]), w2[...]) + b2[...]
    hout[...] = hf
    x = xp[...]
    pout[...] = jnp.concatenate(
        [x[:, c:c + 1] * cw[...] for c in range(3)]
        + [jnp.zeros((BN, 64), F32)], 1)
    aout[...] = _mm(hf, wa[...])
    bout[...] = _mm(hf, wbm[...])


def _edge_in_body(pr, pc, ar, bc, ep, w1e, b1e, w2e, b2e, wef, wd, b1m,
                  w2m, b2m, wphi, bphi, s4):
    d = pr[...] - pc[...]
    dist = (d[:, 0:64] ** 2 + d[:, 64:128] ** 2 + d[:, 128:192] ** 2)
    xe = jnp.concatenate([ep[...], dist], 1)
    ef = _mm(_silu(_mm(xe, w1e[...]) + b1e[...]), w2e[...]) + b2e[...]
    pre = ar[...] + bc[...] + _mm(ef, wef[...]) + _mm(dist, wd[...]) + b1m[...]
    m = _mm(_silu(pre), w2m[...]) + b2m[...]
    phi = _mm(m, wphi[...]) + bphi[...]
    s4[0, :, :] = m[:, 0:128]
    s4[1, :, :] = m[:, 128:256]
    s4[2, :, :] = jnp.concatenate([d[:, 0:64] * phi, d[:, 64:128] * phi], 1)
    s4[3, :, :] = jnp.concatenate([d[:, 128:192] * phi,
                                   jnp.ones((BE, 64), F32)], 1)


def _edge_layer_body(ar, bc, xr, xc, wrad, b1, w2, b2, wphi, bphi, s4):
    d = xr[...] - xc[...]
    rad = (d[:, 0:64] ** 2 + d[:, 64:128] ** 2 + d[:, 128:192] ** 2)
    pre = ar[...] + bc[...] + _mm(rad, wrad[...]) + b1[...]
    m = _mm(_silu(pre), w2[...]) + b2[...]
    phi = _mm(m, wphi[...]) + bphi[...]
    s4[0, :, :] = m[:, 0:128]
    s4[1, :, :] = m[:, 128:256]
    s4[2, :, :] = jnp.concatenate([d[:, 0:64] * phi, d[:, 64:128] * phi], 1)
    s4[3, :, :] = jnp.concatenate([d[:, 128:192] * phi,
                                   jnp.zeros((BE, 64), F32)], 1)


def _edge_out_body(ar, bc, xr, xc, wrad, b1, w2, b2, wp8, ea2, d1_out):
    d = xr[...] - xc[...]
    rad = (d[:, 0:64] ** 2 + d[:, 64:128] ** 2 + d[:, 128:192] ** 2)
    pre = ar[...] + bc[...] + _mm(rad, wrad[...]) + b1[...]
    ea = _mm(_silu(pre), w2[...]) + b2[...]
    ea2[0, :, :] = ea[:, 0:128]
    ea2[1, :, :] = ea[:, 128:256]
    dc0 = _mm(d[:, 0:64], wp8[...])
    dc1 = _mm(d[:, 64:128], wp8[...])
    dc2 = _mm(d[:, 128:192], wp8[...])
    d1_out[...] = dc0 ** 2 + dc1 ** 2 + dc2 ** 2


def _make_fused_layer_body(first):
    BM = 256
    GPB = BM // NPG

    def body(*refs):
        if first:
            (hp, xvp, a4, te, wu1, wu2, bu, ada_w, ada_b, wqkv, bqkv, wo,
             bo, w1, b1, w2, b2, wa, wbm,
             hd_out, xv_out, a_out, b_out, cnt_out) = refs
        else:
            (hp, xvp, a4, cnt_in, te, wu1, wu2, bu, ada_w, ada_b, wqkv,
             bqkv, wo, bo, w1, b1, w2, b2, wa, wbm,
             hd_out, xv_out, a_out, b_out) = refs
        am = jnp.concatenate([a4[0, :, :], a4[1, :, :]], 1)
        ad = jnp.concatenate([a4[2, :, :], a4[3, :, :]], 1)
        hv = hp[...]
        if first:
            cnt = ad[:, 192:193] + 1.0
            cnt_out[...] = jnp.broadcast_to(cnt, (BM, 8))
        else:
            cnt = cnt_in[:, 0:1]
        H = hv + _mm(hv, wu1[...]) + _mm(am, wu2[...]) + bu[...]
        xv_out[...] = xvp[...] + ad / cnt
        modb = _mm(_silu(te[...]), ada_w[...]) + ada_b[...]
        rg = lax.broadcasted_iota(jnp.int32, (BM, GPB), 0) // NPG
        cg = lax.broadcasted_iota(jnp.int32, (BM, GPB), 1)
        R = (rg == cg).astype(F32)
        modx = _mm(R, modb)
        sa = modx[:, 0:256]
        ca = modx[:, 256:512]
        ga = modx[:, 512:768]
        sm = modx[:, 768:1024]
        cm = modx[:, 1024:1280]
        gm = modx[:, 1280:1536]
        h = _ln(H) * (1.0 + ca) + sa
        qkv = _mm(h, wqkv[...]) + bqkv[...]
        rr = lax.broadcasted_iota(jnp.int32, (BM, BM), 0) // NPG
        cc = lax.broadcasted_iota(jnp.int32, (BM, BM), 1) // NPG
        blockmask = rr == cc
        outs = []
        for hh in range(HEADS):
            q = qkv[:, hh * HD:(hh + 1) * HD]
            k = qkv[:, 256 + hh * HD:256 + (hh + 1) * HD]
            v = qkv[:, 512 + hh * HD:512 + (hh + 1) * HD]
            s = lax.dot_general(q, k, (((1,), (1,)), ((), ())),
                                preferred_element_type=F32) * (1.0 / 8.0)
            s = jnp.where(blockmask, s, -1e30)
            e = jnp.exp(s - jnp.max(s, -1, keepdims=True))
            att = e / jnp.sum(e, -1, keepdims=True)
            outs.append(_mm(att, v))
        o = jnp.concatenate(outs, 1)
        H2 = H + ga * (_mm(o, wo[...]) + bo[...])
        h2 = _ln(H2) * (1.0 + cm) + sm
        y = _mm(_silu(_mm(h2, w1[...]) + b1[...]), w2[...]) + b2[...]
        Hd = H2 + gm * y
        hd_out[...] = Hd
        a_out[...] = _mm(Hd, wa[...])
        b_out[...] = _mm(Hd, wbm[...])

    return body


def _upd_proj_body(hp, xvp, a4, cnt_in, wu1, wu2, bu, wa, wbm,
                   hf_out, xv_out, a_out, b_out):
    am = jnp.concatenate([a4[0, :, :], a4[1, :, :]], 1)
    ad = jnp.concatenate([a4[2, :, :], a4[3, :, :]], 1)
    hv = hp[...]
    hf = hv + _mm(hv, wu1[...]) + _mm(am, wu2[...]) + bu[...]
    hf_out[...] = hf
    xv_out[...] = xvp[...] + ad / cnt_in[:, 0:1]
    a_out[...] = _mm(hf, wa[...])
    b_out[...] = _mm(hf, wbm[...])


def _out_node_body(h, xv, agg, wu1, wu2, bu, wp8, wa1, ba1, wa2, ba2,
                   wa, wbm, xcout, hlout, aout, bout):
    hv = h[...]
    h2 = hv + _mm(hv, wu1[...]) + _mm(agg[...], wu2[...]) + bu[...]
    aout[...] = _mm(h2, wa[...])
    bout[...] = _mm(h2, wbm[...])
    x = xv[...]
    xp = jnp.concatenate([_mm(x[:, c * 64:(c + 1) * 64], wp8[...])
                          for c in range(3)], 1)
    rr = lax.broadcasted_iota(jnp.int32, (BN, BN), 0) // NPG
    cc = lax.broadcasted_iota(jnp.int32, (BN, BN), 1) // NPG
    avg = (rr == cc).astype(F32) * (1.0 / NPG)
    xcout[...] = xp - _mm(avg, xp)
    hlout[...] = _mm(_silu(_mm(h2, wa1[...]) + ba1[...]), wa2[...]) + ba2[...]


def _refine_body(a2, b2r, ea, d1, wea, wd18, b1, w2, b2, weh, beh, out):
    eav = jnp.concatenate([ea[0, :, :], ea[1, :, :]], 1)
    pre = (a2[...] + b2r[...] + _mm(eav, wea[...]) + _mm(d1[...], wd18[...])
           + b1[...])
    delta = _mm(_silu(pre), w2[...]) + b2[...]
    eaf = eav + delta
    out[...] = _mm(eaf, weh[...]) + beh[...]


def _call(body, grid, in_specs, out_specs, out_shape):
    return pl.pallas_call(body, grid=grid, in_specs=in_specs,
                          out_specs=out_specs, out_shape=out_shape)


@jax.jit
def kernel(batch, X, H, E_idx, E, t, params):
    row = E_idx[0].astype(jnp.int32)
    col = E_idx[1].astype(jnp.int32)
    p = params

    def wb(lp):
        return lp["w"], _b2(lp["b"])

    ae1w, ae1b = wb(p["atom_emb"]["l1"])
    ae2w, ae2b = wb(p["atom_emb"]["l2"])
    ae1w = jnp.pad(ae1w, ((0, 128 - AC), (0, 0)))
    Hp = jnp.pad(H, ((0, 0), (0, 128 - AC)))
    Xp8 = jnp.pad(X, ((0, 0), (0, 8 - 3)))
    cw = _b2(p["coord_emb_w"])

    ee1w, ee1b = wb(p["edge_emb"]["l1"])
    ee2w, ee2b = wb(p["edge_emb"]["l2"])
    w1e_pad = jnp.zeros((128, DE), F32)
    w1e_pad = w1e_pad.at[0:EC].set(ee1w[0:EC]).at[64:64 + NV].set(ee1w[EC:])
    Epad = jnp.pad(E, ((0, 0), (0, 64 - EC)))

    tm1w, tm1b = wb(p["time_mlp"]["l1"])
    tm2w, tm2b = wb(p["time_mlp"]["l2"])
    freqs = _b2(jnp.exp(-np.log(10000.0)
                        * jnp.arange(D // 2, dtype=F32) / (D // 2)))
    t8 = jnp.broadcast_to(t[:, None], (G, 8))

    im1w, im1b = wb(p["in_msg"]["l1"])
    im2w, im2b = wb(p["in_msg"]["l2"])
    im_wa, im_wb = im1w[0:D], im1w[D:2 * D]
    im_wef, im_wd = im1w[2 * D:2 * D + DE], im1w[2 * D + DE:]
    iu_w, iu_b = wb(p["in_upd"])
    ip_w, ip_b = wb(p["in_phi"])

    wp8 = jnp.broadcast_to(p["coord_pred_w"][:, None], (NV, 8))

    oe1w, oe1b = wb(p["out_edge"]["l1"])
    oe2w, oe2b = wb(p["out_edge"]["l2"])
    ou_w, ou_b = wb(p["out_upd"])
    rf1w, rf1b = wb(p["refine"]["l1"])
    rf2w, rf2b = wb(p["refine"]["l2"])
    rf_wa, rf_wb = rf1w[0:D], rf1w[D:2 * D]
    rf_wea = rf1w[2 * D:2 * D + DE]
    rf_wd18 = jnp.broadcast_to(rf1w[2 * D + DE:2 * D + DE + 1] / 8.0, (8, DE))
    ah1w, ah1b = wb(p["atom_head"]["l1"])
    ah2w, ah2b = wb(p["atom_head"]["l2"])
    eh_w, eh_b = wb(p["edge_head"])
    eh_wp = jnp.pad(eh_w, ((0, 0), (0, 8 - EC)))
    eh_bp = jnp.pad(eh_b, ((0, 0), (0, 8 - EC)))

    gn = N // BN
    ge = EDG // BE

    te_g = _call(
        _te_body, (1,),
        [_rep((G, 8)), _rep((1, 128)), _rep((D, D)), _rep((1, D)),
         _rep((D, D)), _rep((1, D))],
        _rep((G, D)), jax.ShapeDtypeStruct((G, D), F32),
    )(t8, freqs, tm1w, tm1b, tm2w, tm2b)

    Hf, pos, A, B = _call(
        _prep_body, (gn,),
        [_rows(BN, 128), _rows(BN, 8), _rep((128, D)), _rep((1, D)),
         _rep((D, D)), _rep((1, D)), _rep((1, NV)), _rep((D, D)),
         _rep((D, D))],
        [_rows(BN, D), _rows(BN, 256), _rows(BN, D), _rows(BN, D)],
        [jax.ShapeDtypeStruct((N, D), F32),
         jax.ShapeDtypeStruct((N, 256), F32),
         jax.ShapeDtypeStruct((N, D), F32),
         jax.ShapeDtypeStruct((N, D), F32)],
    )(Hp, Xp8, ae1w, ae1b, ae2w, ae2b, cw, im_wa, im_wb)

    Ar, Bc, Pr, Pc = _gather_kernel((D, D, 256, 256), (0, 1, 0, 1),
                                    32)(A, B, pos, pos, row, col)
    S4 = _call(
        _edge_in_body, (ge,),
        [_rows(BE, 256), _rows(BE, 256), _rows(BE, D), _rows(BE, D),
         _rows(BE, 64), _rep((128, DE)), _rep((1, DE)), _rep((DE, DE)),
         _rep((1, DE)), _rep((DE, D)), _rep((NV, D)), _rep((1, D)),
         _rep((D, D)), _rep((1, D)), _rep((D, NV)), _rep((1, NV))],
        pl.BlockSpec((4, BE, 128), lambda i: (0, i, 0)),
        jax.ShapeDtypeStruct((4, EDG, 128), F32),
    )(Pr, Pc, Ar, Bc, Epad, w1e_pad, ee1b, ee2w, ee2b, im_wef, im_wd,
      im1b, im2w, im2b, ip_w, ip_b)
    agg4 = _seg_sum4(S4, row)

    HdPrev = Hf
    XvPrev = pos
    cnt = None
    for i in range(LAYERS):
        dp = p["dit"][i]
        ep = p["egnn"][i]
        ada_w, ada_b = wb(dp["ada"])
        qkv_w, qkv_b = wb(dp["qkv"])
        wo_w, wo_b = wb(dp["wo"])
        dm1w, dm1b = wb(dp["mlp"]["l1"])
        dm2w, dm2b = wb(dp["mlp"]["l2"])
        mg1w, mg1b = wb(ep["msg"]["l1"])
        mg2w, mg2b = wb(ep["msg"]["l2"])
        up_w, up_b = wb(ep["upd"])
        ph_w, ph_b = wb(ep["phi"])
        if i == 0:
            uw, ub = iu_w, iu_b
        else:
            uw, ub = wb(p["egnn"][i - 1]["upd"])
            uw, ub = uw, ub
        first = i == 0
        ins = [HdPrev, XvPrev, agg4] + ([] if first else [cnt]) + [
            te_g, uw[0:D], uw[D:], ub, ada_w, ada_b, qkv_w, qkv_b,
            wo_w, wo_b, dm1w, dm1b, dm2w, dm2b, mg1w[0:D], mg1w[D:2 * D]]
        in_specs = ([_rows(256, D), _rows(256, 256),
                     pl.BlockSpec((4, 256, 128), lambda i: (0, i, 0))]
                    + ([] if first else [_rows(256, 8)])
                    + [pl.BlockSpec((8, D), lambda i: (i, 0)),
                       _rep((D, D)), _rep((D, D)), _rep((1, D)),
                       _rep((D, 6 * D)), _rep((1, 6 * D)),
                       _rep((D, 3 * D)), _rep((1, 3 * D)),
                       _rep((D, D)), _rep((1, D)),
                       _rep((D, 4 * D)), _rep((1, 4 * D)),
                       _rep((4 * D, D)), _rep((1, D)),
                       _rep((D, D)), _rep((D, D))])
        out_specs = [_rows(256, D), _rows(256, 256), _rows(256, D),
                     _rows(256, D)] + ([_rows(256, 8)] if first else [])
        out_shapes = [jax.ShapeDtypeStruct((N, D), F32),
                      jax.ShapeDtypeStruct((N, 256), F32),
                      jax.ShapeDtypeStruct((N, D), F32),
                      jax.ShapeDtypeStruct((N, D), F32)] + (
                          [jax.ShapeDtypeStruct((N, 8), F32)]
                          if first else [])
        outs = _call(_make_fused_layer_body(first), (N // 256,),
                     in_specs, out_specs, out_shapes)(*ins)
        if first:
            Hd, Xv, A, B, cnt = outs
        else:
            Hd, Xv, A, B = outs
        Ar, Bc, Xr, Xc = _gather_kernel((D, D, 256, 256), (0, 1, 0, 1),
                                        32)(A, B, Xv, Xv, row, col)
        S4 = _call(
            _edge_layer_body, (ge,),
            [_rows(BE, D), _rows(BE, D), _rows(BE, 256), _rows(BE, 256),
             _rep((NV, D)), _rep((1, D)), _rep((D, D)), _rep((1, D)),
             _rep((D, NV)), _rep((1, NV))],
            pl.BlockSpec((4, BE, 128), lambda i: (0, i, 0)),
            jax.ShapeDtypeStruct((4, EDG, 128), F32),
        )(Ar, Bc, Xr, Xc, mg1w[2 * D:], mg1b, mg2w, mg2b, ph_w, ph_b)
        agg4 = _seg_sum4(S4, row)
        HdPrev = Hd
        XvPrev = Xv

    up4_w, up4_b = wb(p["egnn"][LAYERS - 1]["upd"])
    Hf, Xv, A, B = _call(
        _upd_proj_body, (gn,),
        [_rows(BN, D), _rows(BN, 256),
         pl.BlockSpec((4, BN, 128), lambda i: (0, i, 0)), _rows(BN, 8),
         _rep((D, D)), _rep((D, D)), _rep((1, D)), _rep((D, D)),
         _rep((D, D))],
        [_rows(BN, D), _rows(BN, 256), _rows(BN, D), _rows(BN, D)],
        [jax.ShapeDtypeStruct((N, D), F32),
         jax.ShapeDtypeStruct((N, 256), F32),
         jax.ShapeDtypeStruct((N, D), F32),
         jax.ShapeDtypeStruct((N, D), F32)],
    )(HdPrev, XvPrev, agg4, cnt, up4_w[0:D], up4_w[D:], up4_b,
      oe1w[0:D], oe1w[D:2 * D])
    Ar, Bc, Xr, Xc = _gather_kernel((D, D, 256, 256), (0, 1, 0, 1),
                                    32)(A, B, Xv, Xv, row, col)
    ea2, d1 = _call(
        _edge_out_body, (ge,),
        [_rows(BE, D), _rows(BE, D), _rows(BE, 256), _rows(BE, 256),
         _rep((NV, DE)), _rep((1, DE)), _rep((DE, DE)), _rep((1, DE)),
         _rep((NV, 8))],
        [pl.BlockSpec((2, BE, 128), lambda i: (0, i, 0)), _rows(BE, 8)],
        [jax.ShapeDtypeStruct((2, EDG, 128), F32),
         jax.ShapeDtypeStruct((EDG, 8), F32)],
    )(Ar, Bc, Xr, Xc, oe1w[2 * D:], oe1b, oe2w, oe2b, wp8)
    agg = _seg_sum2(ea2, row)
    xc24, hlog, A2, B2 = _call(
        _out_node_body, (gn,),
        [_rows(BN, D), _rows(BN, 256), _rows(BN, D), _rep((D, D)),
         _rep((D, D)), _rep((1, D)), _rep((NV, 8)), _rep((D, D)),
         _rep((1, D)), _rep((D, AC)), _rep((1, AC)), _rep((D, D)),
         _rep((D, D))],
        [_rows(BN, 24), _rows(BN, AC), _rows(BN, D), _rows(BN, D)],
        [jax.ShapeDtypeStruct((N, 24), F32),
         jax.ShapeDtypeStruct((N, AC), F32),
         jax.ShapeDtypeStruct((N, D), F32),
         jax.ShapeDtypeStruct((N, D), F32)],
    )(Hf, Xv, agg, ou_w[0:D], ou_w[D:], ou_b, wp8, ah1w, ah1b, ah2w, ah2b,
      rf_wa, rf_wb)

    A2r, B2c = _gather_kernel((D, D), (0, 1), 64)(A2, B2, row, col)
    elog = _call(
        _refine_body, (ge,),
        [_rows(BE, D), _rows(BE, D),
         pl.BlockSpec((2, BE, 128), lambda i: (0, i, 0)), _rows(BE, 8),
         _rep((DE, DE)), _rep((8, DE)), _rep((1, DE)), _rep((DE, DE)),
         _rep((1, DE)), _rep((DE, 8)), _rep((1, 8))],
        _rows(BE, 8), jax.ShapeDtypeStruct((EDG, 8), F32),
    )(A2r, B2c, ea2, d1, rf_wea, rf_wd18, rf1b, rf2w, rf2b, eh_wp, eh_bp)

    x = jnp.stack([xc24[:, 0], xc24[:, 8], xc24[:, 16]], 1)
    return x, hlog, elog[:, :EC]


# speedup vs baseline: 1.1825x; 1.1003x over previous
import functools

import numpy as np
import jax
import jax.numpy as jnp
from jax import lax
from jax.experimental import pallas as pl
from jax.experimental.pallas import tpu as pltpu
from jax.experimental.pallas import tpu_sc as plsc

N = 8192
G = 256
NPG = 32
EDG = 32768
D = 256
DE = 256
NV = 64
AC = 16
EC = 5
HEADS = 4
LAYERS = 4
HD = D // HEADS

NC = 2
NS = 16
NW = NC * NS

BN = 512
BE = 512

F32 = jnp.float32


def _mm(a, b):
    return lax.dot_general(a, b, (((1,), (0,)), ((), ())),
                           preferred_element_type=F32)


def _ln(x):
    m = jnp.mean(x, -1, keepdims=True)
    v = jnp.mean((x - m) ** 2, -1, keepdims=True)
    return (x - m) * lax.rsqrt(v + 1e-6)


def _silu(x):
    return x * jax.nn.sigmoid(x)


def _b2(x):
    return x.reshape(1, -1)


def _pack16(x):
    u = lax.bitcast_convert_type(x, jnp.uint32)
    hi = u[:, 0:128] & jnp.uint32(0xFFFF0000)
    lo = u[:, 128:256] >> 16
    return lax.bitcast_convert_type(hi | lo, F32)


def _unpack16(pk):
    u = lax.bitcast_convert_type(pk, jnp.uint32)
    hi = lax.bitcast_convert_type(u & jnp.uint32(0xFFFF0000), F32)
    lo = lax.bitcast_convert_type(u << 16, F32)
    return jnp.concatenate([hi, lo], 1)


_MESH = plsc.VectorSubcoreMesh(core_axis_name="c", subcore_axis_name="s",
                               num_cores=NC, num_subcores=NS)


@functools.cache
def _gather_kernel(widths, srcs, ch):
    CH = ch
    k = len(widths)
    perw = EDG // NW
    nch = perw // CH
    out_type = [jax.ShapeDtypeStruct((EDG, w), F32) for w in widths]
    scratch = ([pltpu.VMEM((perw,), jnp.int32), pltpu.VMEM((perw,), jnp.int32)]
               + [pltpu.VMEM((CH, w), F32) for w in widths for _ in (0, 1)]
               + [pltpu.SemaphoreType.DMA] * 4)

    def body(*refs):
        tabs = refs[:k]
        rowi, coli = refs[k], refs[k + 1]
        outs = refs[k + 2:2 * k + 2]
        irb, icb = refs[2 * k + 2], refs[2 * k + 3]
        bufs = refs[2 * k + 4:2 * k + 4 + 2 * k]
        gsem = refs[4 * k + 4:4 * k + 6]
        osem = refs[4 * k + 6:4 * k + 8]
        wid = lax.axis_index("s") * NC + lax.axis_index("c")
        e0 = pl.multiple_of(wid * perw, perw)
        pltpu.sync_copy(rowi.at[pl.ds(e0, perw)], irb)
        pltpu.sync_copy(coli.at[pl.ds(e0, perw)], icb)

        def iref(t, j):
            b = irb if srcs[t] == 0 else icb
            return b.at[pl.ds(j * CH, CH)]

        def issue_gathers(j, s):
            for t in range(k):
                pltpu.async_copy(tabs[t].at[iref(t, j)], bufs[2 * t + s],
                                 gsem[s])

        def wait_gathers(s):
            for t in range(k):
                pltpu.make_async_copy(tabs[t].at[iref(t, 0)],
                                      bufs[2 * t + s], gsem[s]).wait()

        def issue_outs(j, s):
            for t in range(k):
                pltpu.async_copy(bufs[2 * t + s],
                                 outs[t].at[pl.ds(e0 + j * CH, CH)], osem[s])

        def wait_outs(s):
            for t in range(k):
                pltpu.make_async_copy(bufs[2 * t + s],
                                      outs[t].at[pl.ds(e0, CH)],
                                      osem[s]).wait()

        issue_gathers(0, 0)

        def pair(t2, carry):
            for u in (0, 1):
                s, s2 = u, 1 - u
                j = 2 * t2 + u

                @pl.when(j >= 1)
                def _():
                    wait_outs(s2)

                @pl.when(j + 1 < nch)
                def _():
                    issue_gathers(j + 1, s2)

                wait_gathers(s)
                issue_outs(j, s)
            return carry

        lax.fori_loop(0, nch // 2, pair, 0)
        wait_outs(1)

    return pl.kernel(body, out_type=out_type, mesh=_MESH,
                     scratch_types=scratch)


@functools.cache
def _scatter_kernel(ngrp):
    CH = 128
    WC = 128
    gpc = ngrp // 2
    rps = N // NS
    eps = EDG // NS
    nch = eps // CH

    out_type = jax.ShapeDtypeStruct((ngrp, N, WC), F32)
    scratch = [
        pltpu.VMEM((CH,), jnp.int32), pltpu.VMEM((CH,), jnp.int32),
        pltpu.VMEM((CH, WC), F32), pltpu.VMEM((CH, WC), F32),
        pltpu.VMEM_SHARED((N, WC), F32),
        pltpu.SemaphoreType.DMA, pltpu.SemaphoreType.DMA,
    ]

    def body(src, idx, zin, out, iv0, iv1, sb0, sb1, acc, ls0, ls1):
        iv = (iv0, iv1)
        sb = (sb0, sb1)
        ls = (ls0, ls1)
        ci = lax.axis_index("c")
        sid = lax.axis_index("s")
        r0 = pl.multiple_of(sid * rps, rps)
        e0 = pl.multiple_of(sid * eps, eps)

        for gp in range(gpc):
            g = ci * gpc + gp

            def issue_load(j, s):
                pltpu.async_copy(idx.at[pl.ds(e0 + j * CH, CH)], iv[s],
                                 ls[s])
                pltpu.async_copy(src.at[g, pl.ds(e0 + j * CH, CH)], sb[s],
                                 ls[s])

            def wait_load(s):
                pltpu.make_async_copy(idx.at[pl.ds(e0, CH)], iv[s],
                                      ls[s]).wait()
                pltpu.make_async_copy(src.at[g, pl.ds(e0, CH)], sb[s],
                                      ls[s]).wait()

            issue_load(0, 0)
            pltpu.sync_copy(zin, acc.at[pl.ds(r0, rps)])
            plsc.subcore_barrier()

            def pair(t2, carry):
                for u in (0, 1):
                    s, s2 = u, 1 - u
                    j = 2 * t2 + u

                    @pl.when(j + 1 < nch)
                    def _():
                        issue_load(j + 1, s2)

                    wait_load(s)
                    pltpu.sync_copy(sb[s], acc.at[iv[s]], add=True)
                return carry

            lax.fori_loop(0, nch // 2, pair, 0)
            plsc.subcore_barrier()
            pltpu.sync_copy(acc.at[pl.ds(r0, rps)],
                            out.at[g, pl.ds(r0, rps)])

    return pl.kernel(body, out_type=out_type, mesh=_MESH,
                     scratch_types=scratch)


def _seg_sum4(src4, idx):
    zin = jnp.zeros((N // NS, 128), F32)
    return _scatter_kernel(4)(src4, idx, zin)


def _seg_sum2(src2, idx):
    zin = jnp.zeros((N // NS, 128), F32)
    out = _scatter_kernel(2)(src2, idx, zin)
    return jnp.concatenate([out[0], out[1]], 1)


def _rows(bm, w):
    return pl.BlockSpec((bm, w), lambda i: (i, 0))


def _rep(shape):
    return pl.BlockSpec(shape, lambda i: tuple(0 for _ in shape))


def _te_body(t8, freqs, w1, b1, w2, b2, out):
    args = t8[:, 0:1] * freqs[...]
    te0 = jnp.concatenate([jnp.cos(args), jnp.sin(args)], 1)
    out[...] = _mm(_silu(_mm(te0, w1[...]) + b1[...]), w2[...]) + b2[...]


def _prep_body(hp, xp, w1, b1, w2, b2, cw, wa, wbm, hout, pout, aout, bout):
    hf = _mm(_silu(_mm(hp[...], w1[...]) + b1[...]), w2[...]) + b2[...]
    hout[...] = hf
    x = xp[...]
    pout[...] = jnp.concatenate(
        [x[:, c:c + 1] * cw[...] for c in range(3)]
        + [jnp.zeros((BN, 64), F32)], 1)
    aout[...] = _pack16(_mm(hf, wa[...]))
    bout[...] = _pack16(_mm(hf, wbm[...]))


def _edge_in_body(pr, pc, ar, bc, ep, w1e, b1e, w2e, b2e, wef, wd, b1m,
                  w2m, b2m, wphi, bphi, s4):
    d = pr[...] - pc[...]
    dist = (d[:, 0:64] ** 2 + d[:, 64:128] ** 2 + d[:, 128:192] ** 2)
    xe = jnp.concatenate([ep[...], dist], 1)
    ef = _mm(_silu(_mm(xe, w1e[...]) + b1e[...]), w2e[...]) + b2e[...]
    pre = (_unpack16(ar[...]) + _unpack16(bc[...]) + _mm(ef, wef[...])
           + _mm(dist, wd[...]) + b1m[...])
    m = _mm(_silu(pre), w2m[...]) + b2m[...]
    phi = _mm(m, wphi[...]) + bphi[...]
    s4[0, :, :] = m[:, 0:128]
    s4[1, :, :] = m[:, 128:256]
    s4[2, :, :] = jnp.concatenate([d[:, 0:64] * phi, d[:, 64:128] * phi], 1)
    s4[3, :, :] = jnp.concatenate([d[:, 128:192] * phi,
                                   jnp.ones((BE, 64), F32)], 1)


def _edge_layer_body(ar, bc, xr, xc, wrad, b1, w2, b2, wphi, bphi, s4):
    d = xr[...] - xc[...]
    rad = (d[:, 0:64] ** 2 + d[:, 64:128] ** 2 + d[:, 128:192] ** 2)
    pre = (_unpack16(ar[...]) + _unpack16(bc[...])
           + _mm(rad, wrad[...]) + b1[...])
    m = _mm(_silu(pre), w2[...]) + b2[...]
    phi = _mm(m, wphi[...]) + bphi[...]
    s4[0, :, :] = m[:, 0:128]
    s4[1, :, :] = m[:, 128:256]
    s4[2, :, :] = jnp.concatenate([d[:, 0:64] * phi, d[:, 64:128] * phi], 1)
    s4[3, :, :] = jnp.concatenate([d[:, 128:192] * phi,
                                   jnp.zeros((BE, 64), F32)], 1)


def _edge_out_body(ar, bc, xr, xc, wrad, b1, w2, b2, wp8, ea2, d1_out):
    d = xr[...] - xc[...]
    rad = (d[:, 0:64] ** 2 + d[:, 64:128] ** 2 + d[:, 128:192] ** 2)
    pre = (_unpack16(ar[...]) + _unpack16(bc[...])
           + _mm(rad, wrad[...]) + b1[...])
    ea = _mm(_silu(pre), w2[...]) + b2[...]
    ea2[0, :, :] = ea[:, 0:128]
    ea2[1, :, :] = ea[:, 128:256]
    dc0 = _mm(d[:, 0:64], wp8[...])
    dc1 = _mm(d[:, 64:128], wp8[...])
    dc2 = _mm(d[:, 128:192], wp8[...])
    d1_out[...] = dc0 ** 2 + dc1 ** 2 + dc2 ** 2


def _make_fused_layer_body(first):
    BM = 256
    GPB = BM // NPG

    def body(*refs):
        if first:
            (hp, xvp, a4, te, wu1, wu2, bu, ada_w, ada_b, wqkv, bqkv, wo,
             bo, w1, b1, w2, b2, wa, wbm,
             hd_out, xv_out, a_out, b_out, cnt_out) = refs
        else:
            (hp, xvp, a4, cnt_in, te, wu1, wu2, bu, ada_w, ada_b, wqkv,
             bqkv, wo, bo, w1, b1, w2, b2, wa, wbm,
             hd_out, xv_out, a_out, b_out) = refs
        am = jnp.concatenate([a4[0, :, :], a4[1, :, :]], 1)
        ad = jnp.concatenate([a4[2, :, :], a4[3, :, :]], 1)
        hv = hp[...]
        if first:
            cnt = ad[:, 192:193] + 1.0
            cnt_out[...] = jnp.broadcast_to(cnt, (BM, 8))
        else:
            cnt = cnt_in[:, 0:1]
        H = hv + _mm(hv, wu1[...]) + _mm(am, wu2[...]) + bu[...]
        xv_out[...] = xvp[...] + ad / cnt
        modb = _mm(_silu(te[...]), ada_w[...]) + ada_b[...]
        rg = lax.broadcasted_iota(jnp.int32, (BM, GPB), 0) // NPG
        cg = lax.broadcasted_iota(jnp.int32, (BM, GPB), 1)
        R = (rg == cg).astype(F32)
        modx = _mm(R, modb)
        sa = modx[:, 0:256]
        ca = modx[:, 256:512]
        ga = modx[:, 512:768]
        sm = modx[:, 768:1024]
        cm = modx[:, 1024:1280]
        gm = modx[:, 1280:1536]
        h = _ln(H) * (1.0 + ca) + sa
        qkv = _mm(h, wqkv[...]) + bqkv[...]
        rr = lax.broadcasted_iota(jnp.int32, (BM, BM), 0) // NPG
        cc = lax.broadcasted_iota(jnp.int32, (BM, BM), 1) // NPG
        blockmask = rr == cc
        outs = []
        for hh in range(HEADS):
            q = qkv[:, hh * HD:(hh + 1) * HD]
            k = qkv[:, 256 + hh * HD:256 + (hh + 1) * HD]
            v = qkv[:, 512 + hh * HD:512 + (hh + 1) * HD]
            s = lax.dot_general(q, k, (((1,), (1,)), ((), ())),
                                preferred_element_type=F32) * (1.0 / 8.0)
            s = jnp.where(blockmask, s, -1e30)
            e = jnp.exp(s - jnp.max(s, -1, keepdims=True))
            att = e / jnp.sum(e, -1, keepdims=True)
            outs.append(_mm(att, v))
        o = jnp.concatenate(outs, 1)
        H2 = H + ga * (_mm(o, wo[...]) + bo[...])
        h2 = _ln(H2) * (1.0 + cm) + sm
        y = _mm(_silu(_mm(h2, w1[...]) + b1[...]), w2[...]) + b2[...]
        Hd = H2 + gm * y
        hd_out[...] = Hd
        a_out[...] = _pack16(_mm(Hd, wa[...]))
        b_out[...] = _pack16(_mm(Hd, wbm[...]))

    return body


def _upd_proj_body(hp, xvp, a4, cnt_in, wu1, wu2, bu, wa, wbm,
                   hf_out, xv_out, a_out, b_out):
    am = jnp.concatenate([a4[0, :, :], a4[1, :, :]], 1)
    ad = jnp.concatenate([a4[2, :, :], a4[3, :, :]], 1)
    hv = hp[...]
    hf = hv + _mm(hv, wu1[...]) + _mm(am, wu2[...]) + bu[...]
    hf_out[...] = hf
    xv_out[...] = xvp[...] + ad / cnt_in[:, 0:1]
    a_out[...] = _pack16(_mm(hf, wa[...]))
    b_out[...] = _pack16(_mm(hf, wbm[...]))


def _out_node_body(h, xv, agg, wu1, wu2, bu, wp8, wa1, ba1, wa2, ba2,
                   wa, wbm, xcout, hlout, aout, bout):
    hv = h[...]
    h2 = hv + _mm(hv, wu1[...]) + _mm(agg[...], wu2[...]) + bu[...]
    aout[...] = _pack16(_mm(h2, wa[...]))
    bout[...] = _pack16(_mm(h2, wbm[...]))
    x = xv[...]
    xp = jnp.concatenate([_mm(x[:, c * 64:(c + 1) * 64], wp8[...])
                          for c in range(3)], 1)
    rr = lax.broadcasted_iota(jnp.int32, (BN, BN), 0) // NPG
    cc = lax.broadcasted_iota(jnp.int32, (BN, BN), 1) // NPG
    avg = (rr == cc).astype(F32) * (1.0 / NPG)
    xcout[...] = xp - _mm(avg, xp)
    hlout[...] = _mm(_silu(_mm(h2, wa1[...]) + ba1[...]), wa2[...]) + ba2[...]


def _refine_body(a2, b2r, ea, d1, wea, wd18, b1, w2, b2, weh, beh, out):
    eav = jnp.concatenate([ea[0, :, :], ea[1, :, :]], 1)
    pre = (_unpack16(a2[...]) + _unpack16(b2r[...]) + _mm(eav, wea[...])
           + _mm(d1[...], wd18[...]) + b1[...])
    delta = _mm(_silu(pre), w2[...]) + b2[...]
    eaf = eav + delta
    out[...] = _mm(eaf, weh[...]) + beh[...]


def _call(body, grid, in_specs, out_specs, out_shape):
    return pl.pallas_call(body, grid=grid, in_specs=in_specs,
                          out_specs=out_specs, out_shape=out_shape)


@jax.jit
def kernel(batch, X, H, E_idx, E, t, params):
    row = E_idx[0].astype(jnp.int32)
    col = E_idx[1].astype(jnp.int32)
    p = params

    def wb(lp):
        return lp["w"], _b2(lp["b"])

    ae1w, ae1b = wb(p["atom_emb"]["l1"])
    ae2w, ae2b = wb(p["atom_emb"]["l2"])
    ae1w = jnp.pad(ae1w, ((0, 128 - AC), (0, 0)))
    Hp = jnp.pad(H, ((0, 0), (0, 128 - AC)))
    Xp8 = jnp.pad(X, ((0, 0), (0, 8 - 3)))
    cw = _b2(p["coord_emb_w"])

    ee1w, ee1b = wb(p["edge_emb"]["l1"])
    ee2w, ee2b = wb(p["edge_emb"]["l2"])
    w1e_pad = jnp.zeros((128, DE), F32)
    w1e_pad = w1e_pad.at[0:EC].set(ee1w[0:EC]).at[64:64 + NV].set(ee1w[EC:])
    Epad = jnp.pad(E, ((0, 0), (0, 64 - EC)))

    tm1w, tm1b = wb(p["time_mlp"]["l1"])
    tm2w, tm2b = wb(p["time_mlp"]["l2"])
    freqs = _b2(jnp.exp(-np.log(10000.0)
                        * jnp.arange(D // 2, dtype=F32) / (D // 2)))
    t8 = jnp.broadcast_to(t[:, None], (G, 8))

    im1w, im1b = wb(p["in_msg"]["l1"])
    im2w, im2b = wb(p["in_msg"]["l2"])
    im_wa, im_wb = im1w[0:D], im1w[D:2 * D]
    im_wef, im_wd = im1w[2 * D:2 * D + DE], im1w[2 * D + DE:]
    iu_w, iu_b = wb(p["in_upd"])
    ip_w, ip_b = wb(p["in_phi"])

    wp8 = jnp.broadcast_to(p["coord_pred_w"][:, None], (NV, 8))

    oe1w, oe1b = wb(p["out_edge"]["l1"])
    oe2w, oe2b = wb(p["out_edge"]["l2"])
    ou_w, ou_b = wb(p["out_upd"])
    rf1w, rf1b = wb(p["refine"]["l1"])
    rf2w, rf2b = wb(p["refine"]["l2"])
    rf_wa, rf_wb = rf1w[0:D], rf1w[D:2 * D]
    rf_wea = rf1w[2 * D:2 * D + DE]
    rf_wd18 = jnp.broadcast_to(rf1w[2 * D + DE:2 * D + DE + 1] / 8.0, (8, DE))
    ah1w, ah1b = wb(p["atom_head"]["l1"])
    ah2w, ah2b = wb(p["atom_head"]["l2"])
    eh_w, eh_b = wb(p["edge_head"])
    eh_wp = jnp.pad(eh_w, ((0, 0), (0, 8 - EC)))
    eh_bp = jnp.pad(eh_b, ((0, 0), (0, 8 - EC)))

    gn = N // BN
    ge = EDG // BE

    te_g = _call(
        _te_body, (1,),
        [_rep((G, 8)), _rep((1, 128)), _rep((D, D)), _rep((1, D)),
         _rep((D, D)), _rep((1, D))],
        _rep((G, D)), jax.ShapeDtypeStruct((G, D), F32),
    )(t8, freqs, tm1w, tm1b, tm2w, tm2b)

    Hf, pos, A, B = _call(
        _prep_body, (gn,),
        [_rows(BN, 128), _rows(BN, 8), _rep((128, D)), _rep((1, D)),
         _rep((D, D)), _rep((1, D)), _rep((1, NV)), _rep((D, D)),
         _rep((D, D))],
        [_rows(BN, D), _rows(BN, 256), _rows(BN, 128), _rows(BN, 128)],
        [jax.ShapeDtypeStruct((N, D), F32),
         jax.ShapeDtypeStruct((N, 256), F32),
         jax.ShapeDtypeStruct((N, 128), F32),
         jax.ShapeDtypeStruct((N, 128), F32)],
    )(Hp, Xp8, ae1w, ae1b, ae2w, ae2b, cw, im_wa, im_wb)

    Ar, Bc, Pr, Pc = _gather_kernel((128, 128, 256, 256), (0, 1, 0, 1),
                                    64)(A, B, pos, pos, row, col)
    S4 = _call(
        _edge_in_body, (ge,),
        [_rows(BE, 256), _rows(BE, 256), _rows(BE, 128), _rows(BE, 128),
         _rows(BE, 64), _rep((128, DE)), _rep((1, DE)), _rep((DE, DE)),
         _rep((1, DE)), _rep((DE, D)), _rep((NV, D)), _rep((1, D)),
         _rep((D, D)), _rep((1, D)), _rep((D, NV)), _rep((1, NV))],
        pl.BlockSpec((4, BE, 128), lambda i: (0, i, 0)),
        jax.ShapeDtypeStruct((4, EDG, 128), F32),
    )(Pr, Pc, Ar, Bc, Epad, w1e_pad, ee1b, ee2w, ee2b, im_wef, im_wd,
      im1b, im2w, im2b, ip_w, ip_b)
    agg4 = _seg_sum4(S4, row)

    HdPrev = Hf
    XvPrev = pos
    cnt = None
    for i in range(LAYERS):
        dp = p["dit"][i]
        ep = p["egnn"][i]
        ada_w, ada_b = wb(dp["ada"])
        qkv_w, qkv_b = wb(dp["qkv"])
        wo_w, wo_b = wb(dp["wo"])
        dm1w, dm1b = wb(dp["mlp"]["l1"])
        dm2w, dm2b = wb(dp["mlp"]["l2"])
        mg1w, mg1b = wb(ep["msg"]["l1"])
        mg2w, mg2b = wb(ep["msg"]["l2"])
        up_w, up_b = wb(ep["upd"])
        ph_w, ph_b = wb(ep["phi"])
        if i == 0:
            uw, ub = iu_w, iu_b
        else:
            uw, ub = wb(p["egnn"][i - 1]["upd"])
            uw, ub = uw, ub
        first = i == 0
        ins = [HdPrev, XvPrev, agg4] + ([] if first else [cnt]) + [
            te_g, uw[0:D], uw[D:], ub, ada_w, ada_b, qkv_w, qkv_b,
            wo_w, wo_b, dm1w, dm1b, dm2w, dm2b, mg1w[0:D], mg1w[D:2 * D]]
        in_specs = ([_rows(256, D), _rows(256, 256),
                     pl.BlockSpec((4, 256, 128), lambda i: (0, i, 0))]
                    + ([] if first else [_rows(256, 8)])
                    + [pl.BlockSpec((8, D), lambda i: (i, 0)),
                       _rep((D, D)), _rep((D, D)), _rep((1, D)),
                       _rep((D, 6 * D)), _rep((1, 6 * D)),
                       _rep((D, 3 * D)), _rep((1, 3 * D)),
                       _rep((D, D)), _rep((1, D)),
                       _rep((D, 4 * D)), _rep((1, 4 * D)),
                       _rep((4 * D, D)), _rep((1, D)),
                       _rep((D, D)), _rep((D, D))])
        out_specs = [_rows(256, D), _rows(256, 256), _rows(256, 128),
                     _rows(256, 128)] + ([_rows(256, 8)] if first else [])
        out_shapes = [jax.ShapeDtypeStruct((N, D), F32),
                      jax.ShapeDtypeStruct((N, 256), F32),
                      jax.ShapeDtypeStruct((N, 128), F32),
                      jax.ShapeDtypeStruct((N, 128), F32)] + (
                          [jax.ShapeDtypeStruct((N, 8), F32)]
                          if first else [])
        outs = _call(_make_fused_layer_body(first), (N // 256,),
                     in_specs, out_specs, out_shapes)(*ins)
        if first:
            Hd, Xv, A, B, cnt = outs
        else:
            Hd, Xv, A, B = outs
        Ar, Bc, Xr, Xc = _gather_kernel((128, 128, 256, 256),
                                        (0, 1, 0, 1), 64)(A, B, Xv, Xv,
                                                          row, col)
        S4 = _call(
            _edge_layer_body, (ge,),
            [_rows(BE, 128), _rows(BE, 128), _rows(BE, 256), _rows(BE, 256),
             _rep((NV, D)), _rep((1, D)), _rep((D, D)), _rep((1, D)),
             _rep((D, NV)), _rep((1, NV))],
            pl.BlockSpec((4, BE, 128), lambda i: (0, i, 0)),
            jax.ShapeDtypeStruct((4, EDG, 128), F32),
        )(Ar, Bc, Xr, Xc, mg1w[2 * D:], mg1b, mg2w, mg2b, ph_w, ph_b)
        agg4 = _seg_sum4(S4, row)
        HdPrev = Hd
        XvPrev = Xv

    up4_w, up4_b = wb(p["egnn"][LAYERS - 1]["upd"])
    Hf, Xv, A, B = _call(
        _upd_proj_body, (gn,),
        [_rows(BN, D), _rows(BN, 256),
         pl.BlockSpec((4, BN, 128), lambda i: (0, i, 0)), _rows(BN, 8),
         _rep((D, D)), _rep((D, D)), _rep((1, D)), _rep((D, D)),
         _rep((D, D))],
        [_rows(BN, D), _rows(BN, 256), _rows(BN, 128), _rows(BN, 128)],
        [jax.ShapeDtypeStruct((N, D), F32),
         jax.ShapeDtypeStruct((N, 256), F32),
         jax.ShapeDtypeStruct((N, 128), F32),
         jax.ShapeDtypeStruct((N, 128), F32)],
    )(HdPrev, XvPrev, agg4, cnt, up4_w[0:D], up4_w[D:], up4_b,
      oe1w[0:D], oe1w[D:2 * D])
    Ar, Bc, Xr, Xc = _gather_kernel((128, 128, 256, 256), (0, 1, 0, 1),
                                    64)(A, B, Xv, Xv, row, col)
    ea2, d1 = _call(
        _edge_out_body, (ge,),
        [_rows(BE, 128), _rows(BE, 128), _rows(BE, 256), _rows(BE, 256),
         _rep((NV, DE)), _rep((1, DE)), _rep((DE, DE)), _rep((1, DE)),
         _rep((NV, 8))],
        [pl.BlockSpec((2, BE, 128), lambda i: (0, i, 0)), _rows(BE, 8)],
        [jax.ShapeDtypeStruct((2, EDG, 128), F32),
         jax.ShapeDtypeStruct((EDG, 8), F32)],
    )(Ar, Bc, Xr, Xc, oe1w[2 * D:], oe1b, oe2w, oe2b, wp8)
    agg = _seg_sum2(ea2, row)
    xc24, hlog, A2, B2 = _call(
        _out_node_body, (gn,),
        [_rows(BN, D), _rows(BN, 256), _rows(BN, D), _rep((D, D)),
         _rep((D, D)), _rep((1, D)), _rep((NV, 8)), _rep((D, D)),
         _rep((1, D)), _rep((D, AC)), _rep((1, AC)), _rep((D, D)),
         _rep((D, D))],
        [_rows(BN, 24), _rows(BN, AC), _rows(BN, 128), _rows(BN, 128)],
        [jax.ShapeDtypeStruct((N, 24), F32),
         jax.ShapeDtypeStruct((N, AC), F32),
         jax.ShapeDtypeStruct((N, 128), F32),
         jax.ShapeDtypeStruct((N, 128), F32)],
    )(Hf, Xv, agg, ou_w[0:D], ou_w[D:], ou_b, wp8, ah1w, ah1b, ah2w, ah2b,
      rf_wa, rf_wb)

    A2r, B2c = _gather_kernel((128, 128), (0, 1), 128)(A2, B2, row, col)
    elog = _call(
        _refine_body, (ge,),
        [_rows(BE, 128), _rows(BE, 128),
         pl.BlockSpec((2, BE, 128), lambda i: (0, i, 0)), _rows(BE, 8),
         _rep((DE, DE)), _rep((8, DE)), _rep((1, DE)), _rep((DE, DE)),
         _rep((1, DE)), _rep((DE, 8)), _rep((1, 8))],
        _rows(BE, 8), jax.ShapeDtypeStruct((EDG, 8), F32),
    )(A2r, B2c, ea2, d1, rf_wea, rf_wd18, rf1b, rf2w, rf2b, eh_wp, eh_bp)

    x = jnp.stack([xc24[:, 0], xc24[:, 8], xc24[:, 16]], 1)
    return x, hlog, elog[:, :EC]
